# baseline (speedup 1.0000x reference)
import functools

import jax
import jax.numpy as jnp
from jax import lax
from jax.experimental import pallas as pl
from jax.experimental.pallas import tpu as pltpu
from jax.experimental.pallas import tpu_sc as plsc

_ATTR_INDEX = 8

_NC = 2
_NS = 16
_L = 16
_NW = _NC * _NS
_GW = 128


_CH = 2048


def _make_sc_kernel(B, D, C):
    n = B // _NW

    mesh = plsc.VectorSubcoreMesh(core_axis_name="c", subcore_axis_name="s")

    @functools.partial(
        pl.kernel,
        mesh=mesh,
        out_type=jax.ShapeDtypeStruct((B,), jnp.float32),
        compiler_params=pltpu.CompilerParams(
            use_tc_tiling_on_sc=False, needs_layout_passes=False
        ),
        scratch_types=[
            pltpu.VMEM((_CH, D), jnp.float32),
            pltpu.VMEM((n,), jnp.int32),
            pltpu.VMEM((n,), jnp.float32),
            pltpu.SemaphoreType.DMA,
        ],
    )
    def k(z_hbm, a_hbm, out_hbm, zbuf, gidx, vals, sem):
        wid = lax.axis_index("s") * _NC + lax.axis_index("c")
        base = wid * n

        iota = lax.iota(jnp.int32, _L)
        col = jnp.full((_L,), _ATTR_INDEX, jnp.int32)

        def chunk(ch, carry):
            off = ch * _CH
            pltpu.sync_copy(z_hbm.at[pl.ds(base + off, _CH)], zbuf)

            def build(j, c2):
                rows = j * _L + iota
                idxf = plsc.load_gather(zbuf, [rows, col])
                gidx[pl.ds(off + j * _L, _L)] = (
                    (base + off + rows) * C + idxf.astype(jnp.int32)
                )
                return c2

            lax.fori_loop(0, _CH // _L, build, 0)
            return carry

        lax.fori_loop(0, n // _CH, chunk, 0)

        def gather(g, carry):
            pltpu.async_copy(
                a_hbm.at[gidx.at[pl.ds(g * _GW, _GW)]],
                vals.at[pl.ds(g * _GW, _GW)],
                sem,
            ).wait()
            return carry

        lax.fori_loop(0, n // _GW, gather, 0)

        def scale(j, carry):
            vals[pl.ds(j * _L, _L)] = vals[pl.ds(j * _L, _L)] * jnp.float32(0.999)
            return carry

        lax.fori_loop(0, n // _L, scale, 0)

        pltpu.sync_copy(vals, out_hbm.at[pl.ds(base, n)])

    return k


@jax.jit
def kernel(z, a):
    b, c = a.shape
    k = _make_sc_kernel(b, z.shape[1], c)
    return k(z, a.reshape(-1))


# speedup vs baseline: 1.1555x; 1.1555x over previous
import functools

import jax
import jax.numpy as jnp
from jax import lax
from jax.experimental import pallas as pl
from jax.experimental.pallas import tpu as pltpu
from jax.experimental.pallas import tpu_sc as plsc

_ATTR_INDEX = 8

_NC = 2
_NS = 16
_L = 16
_NW = _NC * _NS
_GW = 128
_CH = 1024


def _make_sc_kernel(B, D, C):
    n = B // _NW
    nch = n // _CH

    mesh = plsc.VectorSubcoreMesh(core_axis_name="c", subcore_axis_name="s")

    @functools.partial(
        pl.kernel,
        mesh=mesh,
        out_type=jax.ShapeDtypeStruct((B,), jnp.float32),
        compiler_params=pltpu.CompilerParams(
            use_tc_tiling_on_sc=False, needs_layout_passes=False
        ),
        scratch_types=[
            pltpu.VMEM((2 * _CH, D), jnp.float32),
            pltpu.VMEM((n,), jnp.int32),
            pltpu.VMEM((n,), jnp.float32),
            pltpu.SemaphoreType.DMA,
            pltpu.SemaphoreType.DMA,
        ],
    )
    def k(z_hbm, a_hbm, out_hbm, zbuf, gidx, vals, zsem, gsem):
        wid = lax.axis_index("s") * _NC + lax.axis_index("c")
        base = wid * n

        iota = lax.iota(jnp.int32, _L)
        col = jnp.full((_L,), _ATTR_INDEX, jnp.int32)

        pltpu.async_copy(
            z_hbm.at[pl.ds(base, _CH)], zbuf.at[pl.ds(0, _CH)], zsem
        )
        for ch in range(nch):
            par = (ch % 2) * _CH
            pltpu.make_async_copy(
                z_hbm.at[pl.ds(base + ch * _CH, _CH)],
                zbuf.at[pl.ds(par, _CH)],
                zsem,
            ).wait()
            if ch + 1 < nch:
                pltpu.async_copy(
                    z_hbm.at[pl.ds(base + (ch + 1) * _CH, _CH)],
                    zbuf.at[pl.ds(((ch + 1) % 2) * _CH, _CH)],
                    zsem,
                )

            def build(j, carry, ch=ch, par=par):
                rows = j * _L + iota
                idxf = plsc.load_gather(zbuf, [par + rows, col])
                gidx[pl.ds(ch * _CH + j * _L, _L)] = (
                    (base + ch * _CH + rows) * C + idxf.astype(jnp.int32)
                )
                return carry

            lax.fori_loop(0, _CH // _L, build, 0)

            def fire(g, carry, ch=ch):
                off = ch * _CH + g * _GW
                pltpu.async_copy(
                    a_hbm.at[gidx.at[pl.ds(off, _GW)]],
                    vals.at[pl.ds(off, _GW)],
                    gsem,
                )
                return carry

            lax.fori_loop(0, _CH // _GW, fire, 0)

        pltpu.make_async_copy(a_hbm.at[pl.ds(0, n)], vals, gsem).wait()

        def scale(j, carry):
            vals[pl.ds(j * _L, _L)] = vals[pl.ds(j * _L, _L)] * jnp.float32(0.999)
            return carry

        lax.fori_loop(0, n // _L, scale, 0)

        pltpu.sync_copy(vals, out_hbm.at[pl.ds(base, n)])

    return k


@jax.jit
def kernel(z, a):
    b, c = a.shape
    k = _make_sc_kernel(b, z.shape[1], c)
    return k(z, a.reshape(-1))


# speedup vs baseline: 1.1580x; 1.0021x over previous
import functools

import jax
import jax.numpy as jnp
from jax import lax
from jax.experimental import pallas as pl
from jax.experimental.pallas import tpu as pltpu
from jax.experimental.pallas import tpu_sc as plsc

_ATTR_INDEX = 8

_NC = 2
_NS = 16
_L = 16
_NW = _NC * _NS
_GW = 1024
_CH = 1024


def _make_sc_kernel(B, D, C):
    n = B // _NW
    nch = n // _CH

    mesh = plsc.VectorSubcoreMesh(core_axis_name="c", subcore_axis_name="s")

    @functools.partial(
        pl.kernel,
        mesh=mesh,
        out_type=jax.ShapeDtypeStruct((B,), jnp.float32),
        compiler_params=pltpu.CompilerParams(
            use_tc_tiling_on_sc=False, needs_layout_passes=False
        ),
        scratch_types=[
            pltpu.VMEM((2 * _CH, D), jnp.float32),
            pltpu.VMEM((n,), jnp.int32),
            pltpu.VMEM((n,), jnp.float32),
            pltpu.SemaphoreType.DMA,
            pltpu.SemaphoreType.DMA,
        ],
    )
    def k(z_hbm, a_hbm, out_hbm, zbuf, gidx, vals, zsem, gsem):
        wid = lax.axis_index("s") * _NC + lax.axis_index("c")
        base = wid * n

        iota = lax.iota(jnp.int32, _L)
        col = jnp.full((_L,), _ATTR_INDEX, jnp.int32)

        pltpu.async_copy(
            z_hbm.at[pl.ds(base, _CH)], zbuf.at[pl.ds(0, _CH)], zsem
        )
        for ch in range(nch):
            par = (ch % 2) * _CH
            pltpu.make_async_copy(
                z_hbm.at[pl.ds(base + ch * _CH, _CH)],
                zbuf.at[pl.ds(par, _CH)],
                zsem,
            ).wait()
            if ch + 1 < nch:
                pltpu.async_copy(
                    z_hbm.at[pl.ds(base + (ch + 1) * _CH, _CH)],
                    zbuf.at[pl.ds(((ch + 1) % 2) * _CH, _CH)],
                    zsem,
                )

            def build(j, carry, ch=ch, par=par):
                rows = j * _L + iota
                idxf = plsc.load_gather(zbuf, [par + rows, col])
                gidx[pl.ds(ch * _CH + j * _L, _L)] = (
                    (base + ch * _CH + rows) * C + idxf.astype(jnp.int32)
                )
                return carry

            lax.fori_loop(0, _CH // _L, build, 0)

            def fire(g, carry, ch=ch):
                off = ch * _CH + g * _GW
                pltpu.async_copy(
                    a_hbm.at[gidx.at[pl.ds(off, _GW)]],
                    vals.at[pl.ds(off, _GW)],
                    gsem,
                )
                return carry

            lax.fori_loop(0, _CH // _GW, fire, 0)

        pltpu.make_async_copy(a_hbm.at[pl.ds(0, n)], vals, gsem).wait()

        def scale(j, carry):
            vals[pl.ds(j * _L, _L)] = vals[pl.ds(j * _L, _L)] * jnp.float32(0.999)
            return carry

        lax.fori_loop(0, n // _L, scale, 0)

        pltpu.sync_copy(vals, out_hbm.at[pl.ds(base, n)])

    return k


@jax.jit
def kernel(z, a):
    b, c = a.shape
    k = _make_sc_kernel(b, z.shape[1], c)
    return k(z, a.reshape(-1))
